# Initial kernel scaffold; baseline (speedup 1.0000x reference)
import jax
import jax.numpy as jnp
from jax.experimental import pallas as pl


def kernel(x, edge_index_1_l, edge_index_2_l, edge_index_1_g, edge_index_2_g, batch, params):
    raise NotImplementedError("write your pallas kernel here")


# speedup vs baseline: 2.5775x; 2.5775x over previous
import functools

import jax
import jax.numpy as jnp
from jax import lax
from jax.experimental import pallas as pl
from jax.experimental.pallas import tpu as pltpu
from jax.experimental.pallas import tpu_sc as plsc

_N = 10000
_E = 160000
_F = 652
_D = 128
_G = 64

_NC = 2
_NS = 16
_NW = _NC * _NS

_CH = 128
_ECHUNKS = 40
_EPAD = _NW * _ECHUNKS * _CH
_AROWS = 10240
_DUMP = _N
_RPT = _AROWS // _NS
_ZCH = _RPT // _CH

_BN = 400
_NB = _N // _BN
_PB = 200
_PNB = _N // _PB

_mesh = plsc.VectorSubcoreMesh(core_axis_name="c", subcore_axis_name="s")


@functools.partial(
    pl.kernel,
    out_type=jax.ShapeDtypeStruct((4 * _AROWS, _D), jnp.float32),
    mesh=_mesh,
    scratch_types=[
        pltpu.VMEM((10, _CH), jnp.int32),
        pltpu.VMEM((_CH, _D), jnp.float32),
        pltpu.SemaphoreType.DMA,
    ],
)
def _sc_gather_p1(tab_hbm, idx_hbm, out_hbm, idx_v, rows_v, sem):
    gwid = lax.axis_index("c") * _NS + lax.axis_index("s")
    pltpu.sync_copy(idx_hbm.at[gwid], idx_v)
    for j in range(10):
        pltpu.async_copy(tab_hbm.at[idx_v.at[j]], rows_v, sem).wait()
        pltpu.sync_copy(rows_v, out_hbm.at[pl.ds(gwid * 1280 + j * _CH, _CH)])


@functools.partial(
    pl.kernel,
    out_type=jax.ShapeDtypeStruct((4, _NC, _AROWS, _D), jnp.float32),
    mesh=_mesh,
    scratch_types=[
        pltpu.VMEM((_ECHUNKS, _CH), jnp.int32),
        pltpu.VMEM((_ECHUNKS, _CH), jnp.int32),
        pltpu.VMEM((_CH, _D), jnp.float32),
        pltpu.VMEM((_CH, _D), jnp.float32),
        pltpu.VMEM_SHARED((_AROWS, _D), jnp.float32),
        pltpu.SemaphoreType.DMA,
    ],
)
def _sc_segsum4(p_hbm, src_hbm, dst_hbm, zero_hbm, out_hbm,
                src_v, dst_v, rows_v, zero_v, accum, sem):
    cid = lax.axis_index("c")
    sid = lax.axis_index("s")
    gwid = cid * _NS + sid
    pltpu.sync_copy(zero_hbm, zero_v)
    for conv in range(4):
        for z in range(_ZCH):
            pltpu.sync_copy(zero_v, accum.at[pl.ds(sid * _RPT + z * _CH, _CH)])
        pltpu.sync_copy(src_hbm.at[conv, gwid], src_v)
        pltpu.sync_copy(dst_hbm.at[conv, gwid], dst_v)
        plsc.subcore_barrier()

        def _body(j, carry):
            pltpu.async_copy(p_hbm.at[src_v.at[j]], rows_v, sem).wait()
            pltpu.sync_copy(rows_v, accum.at[dst_v.at[j]], add=True)
            return carry

        lax.fori_loop(0, _ECHUNKS, _body, 0)
        plsc.subcore_barrier()
        for z in range(_ZCH):
            r0 = sid * _RPT + z * _CH
            pltpu.sync_copy(accum.at[pl.ds(r0, _CH)], rows_v)
            pltpu.sync_copy(rows_v, out_hbm.at[conv, cid, pl.ds(r0, _CH)])
        plsc.subcore_barrier()


def _post_kernel(p_ref, a_ref, sc_ref, b1_ref, w2_ref, b2_ref,
                 wm1_ref, bm1_ref, wm2_ref, bm2_ref, r_ref, st_ref):
    i = pl.program_id(0)
    xs = []
    for c in range(4):
        ac = a_ref[c, 0] + a_ref[c, 1]
        u = jnp.maximum(sc_ref[c][None, :] * p_ref[c] + ac + b1_ref[c][None, :], 0.0)
        xc = jnp.dot(u, w2_ref[c], preferred_element_type=jnp.float32)
        xs.append(jnp.maximum(xc + b2_ref[c][None, :], 0.0))
    cat = jnp.concatenate([xs[0], xs[2], xs[1], xs[3]], axis=1)
    y = jnp.maximum(
        jnp.dot(cat, wm1_ref[...], preferred_element_type=jnp.float32) + bm1_ref[...], 0.0)
    r = jnp.dot(y, wm2_ref[...], preferred_element_type=jnp.float32) + bm2_ref[...]
    r_ref[...] = r
    st = jnp.concatenate(
        [jnp.sum(r, axis=0)[None], jnp.sum(r * r, axis=0)[None],
         jnp.zeros((6, _D), jnp.float32)], axis=0)

    @pl.when(i == 0)
    def _():
        st_ref[...] = st

    @pl.when(i > 0)
    def _():
        st_ref[...] = st_ref[...] + st


def _tc_post(P, A, scale, b1, W2, b2, Wm1, bm1, Wm2, bm2):
    return pl.pallas_call(
        _post_kernel,
        grid=(_NB,),
        in_specs=[
            pl.BlockSpec((4, _BN, _D), lambda i: (0, i, 0)),
            pl.BlockSpec((4, _NC, _BN, _D), lambda i: (0, 0, i, 0)),
            pl.BlockSpec((4, _D), lambda i: (0, 0)),
            pl.BlockSpec((4, _D), lambda i: (0, 0)),
            pl.BlockSpec((4, _D, _D), lambda i: (0, 0, 0)),
            pl.BlockSpec((4, _D), lambda i: (0, 0)),
            pl.BlockSpec((4 * _D, _D), lambda i: (0, 0)),
            pl.BlockSpec((1, _D), lambda i: (0, 0)),
            pl.BlockSpec((_D, _D), lambda i: (0, 0)),
            pl.BlockSpec((1, _D), lambda i: (0, 0)),
        ],
        out_specs=[
            pl.BlockSpec((_BN, _D), lambda i: (i, 0)),
            pl.BlockSpec((8, _D), lambda i: (0, 0)),
        ],
        out_shape=[
            jax.ShapeDtypeStruct((_N, _D), jnp.float32),
            jax.ShapeDtypeStruct((8, _D), jnp.float32),
        ],
    )(P, A, scale, b1, W2, b2, Wm1, bm1, Wm2, bm2)


def _bn_next_kernel(r_ref, st_ref, g_ref, bt_ref, wn_ref, h_ref, pn_ref):
    mean = st_ref[0] * (1.0 / _N)
    var = st_ref[1] * (1.0 / _N) - mean * mean
    inv = lax.rsqrt(var + 1e-5) * g_ref[0]
    h = (r_ref[...] - mean[None, :]) * inv[None, :] + bt_ref[...]
    h_ref[...] = h
    for c in range(4):
        pn_ref[c] = jnp.dot(h, wn_ref[c], preferred_element_type=jnp.float32)


def _tc_bn_next(R, st, gamma, beta, Wn):
    return pl.pallas_call(
        _bn_next_kernel,
        grid=(_NB,),
        in_specs=[
            pl.BlockSpec((_BN, _D), lambda i: (i, 0)),
            pl.BlockSpec((8, _D), lambda i: (0, 0)),
            pl.BlockSpec((1, _D), lambda i: (0, 0)),
            pl.BlockSpec((1, _D), lambda i: (0, 0)),
            pl.BlockSpec((4, _D, _D), lambda i: (0, 0, 0)),
        ],
        out_specs=[
            pl.BlockSpec((_BN, _D), lambda i: (i, 0)),
            pl.BlockSpec((4, _BN, _D), lambda i: (0, i, 0)),
        ],
        out_shape=[
            jax.ShapeDtypeStruct((_N, _D), jnp.float32),
            jax.ShapeDtypeStruct((4, _N, _D), jnp.float32),
        ],
    )(R, st, gamma, beta, Wn)


def _bn_only_kernel(r_ref, st_ref, g_ref, bt_ref, h_ref):
    mean = st_ref[0] * (1.0 / _N)
    var = st_ref[1] * (1.0 / _N) - mean * mean
    inv = lax.rsqrt(var + 1e-5) * g_ref[0]
    h_ref[...] = (r_ref[...] - mean[None, :]) * inv[None, :] + bt_ref[...]


def _tc_bn_only(R, st, gamma, beta):
    return pl.pallas_call(
        _bn_only_kernel,
        grid=(_NB,),
        in_specs=[
            pl.BlockSpec((_BN, _D), lambda i: (i, 0)),
            pl.BlockSpec((8, _D), lambda i: (0, 0)),
            pl.BlockSpec((1, _D), lambda i: (0, 0)),
            pl.BlockSpec((1, _D), lambda i: (0, 0)),
        ],
        out_specs=pl.BlockSpec((_BN, _D), lambda i: (i, 0)),
        out_shape=jax.ShapeDtypeStruct((_N, _D), jnp.float32),
    )(R, st, gamma, beta)


def _pool_fc_kernel(h1_ref, h2_ref, h3_ref, h4_ref, b_ref,
                    w1_ref, b1_ref, w2_ref, b2_ref, w3_ref, b3_ref,
                    w4_ref, b4_ref, out_ref, acc_ref, cnt_ref):
    i = pl.program_id(0)

    @pl.when(i == 0)
    def _():
        acc_ref[...] = jnp.zeros_like(acc_ref)
        cnt_ref[...] = jnp.zeros_like(cnt_ref)

    bb = b_ref[...].reshape(_PB)
    gid = lax.broadcasted_iota(jnp.int32, (_G, _PB), 0)
    oh = (bb[None, :] == gid).astype(jnp.float32)
    cat = jnp.concatenate(
        [h1_ref[...], h2_ref[...], h3_ref[...], h4_ref[...]], axis=1)
    acc_ref[...] = acc_ref[...] + jnp.dot(oh, cat, preferred_element_type=jnp.float32)
    cnt_ref[...] = cnt_ref[...] + jnp.broadcast_to(
        jnp.sum(oh, axis=1, keepdims=True), (_G, _D))

    @pl.when(i == _PNB - 1)
    def _():
        cnt = cnt_ref[...][:, :1]
        pooled = acc_ref[...] / jnp.maximum(cnt, 1.0)
        z = jnp.maximum(
            jnp.dot(pooled, w1_ref[...], preferred_element_type=jnp.float32) + b1_ref[...], 0.0)
        z = jnp.maximum(
            jnp.dot(z, w2_ref[...], preferred_element_type=jnp.float32) + b2_ref[...], 0.0)
        z = jnp.maximum(
            jnp.dot(z, w3_ref[...], preferred_element_type=jnp.float32) + b3_ref[...], 0.0)
        out_ref[...] = jnp.dot(z, w4_ref[...], preferred_element_type=jnp.float32) + b4_ref[...]


def _tc_pool_fc(h1, h2, h3, h4, batch_r, w1, b1, w2, b2, w3, b3, w4, b4):
    hspec = pl.BlockSpec((_PB, _D), lambda i: (i, 0))
    return pl.pallas_call(
        _pool_fc_kernel,
        grid=(_PNB,),
        in_specs=[
            hspec, hspec, hspec, hspec,
            pl.BlockSpec((1, 1, _PB), lambda i: (i, 0, 0)),
            pl.BlockSpec((4 * _D, _D), lambda i: (0, 0)),
            pl.BlockSpec((1, _D), lambda i: (0, 0)),
            pl.BlockSpec((_D, _D), lambda i: (0, 0)),
            pl.BlockSpec((1, _D), lambda i: (0, 0)),
            pl.BlockSpec((_D, _D), lambda i: (0, 0)),
            pl.BlockSpec((1, _D), lambda i: (0, 0)),
            pl.BlockSpec((_D, _D), lambda i: (0, 0)),
            pl.BlockSpec((1, _D), lambda i: (0, 0)),
        ],
        out_specs=pl.BlockSpec((_G, _D), lambda i: (0, 0)),
        out_shape=jax.ShapeDtypeStruct((_G, _D), jnp.float32),
        scratch_shapes=[
            pltpu.VMEM((_G, 4 * _D), jnp.float32),
            pltpu.VMEM((_G, _D), jnp.float32),
        ],
    )(h1, h2, h3, h4, batch_r, w1, b1, w2, b2, w3, b3, w4, b4)


_CONVS = ("1_l", "2_l", "1_g", "2_g")


def kernel(x, edge_index_1_l, edge_index_2_l, edge_index_1_g, edge_index_2_g,
           batch, params):
    p = params
    eis = (edge_index_1_l, edge_index_2_l, edge_index_1_g, edge_index_2_g)

    npad = _EPAD - _E
    srcs, dsts = [], []
    for ei in eis:
        srcs.append(jnp.concatenate([ei[0], jnp.zeros((npad,), jnp.int32)]))
        dsts.append(jnp.concatenate(
            [ei[1], jnp.full((npad,), _DUMP, jnp.int32)]).reshape(_NW, _ECHUNKS, _CH))
    dst_all = jnp.stack(dsts)
    src_l1 = jnp.stack(
        [(s + c * _AROWS).reshape(_NW, _ECHUNKS, _CH) for c, s in enumerate(srcs)])
    src_rest = jnp.stack(
        [(s + c * _N).reshape(_NW, _ECHUNKS, _CH) for c, s in enumerate(srcs)])
    zeros2d = jnp.zeros((_CH, _D), jnp.float32)

    l1tab = jnp.concatenate(
        [p["conv1_%s" % t]["nn"]["l1"]["W"] for t in _CONVS], axis=0)
    xpad = jnp.concatenate([x, jnp.zeros((_AROWS - _N,), jnp.int32)])
    xidx = jnp.concatenate([xpad + c * _F for c in range(4)]).reshape(_NW, 10, _CH)
    Pflat = _sc_gather_p1(l1tab, xidx)
    P = Pflat.reshape(4, _AROWS, _D)

    outs = []
    for L in (1, 2, 3, 4):
        cps = [p["conv%d_%s" % (L, t)] for t in _CONVS]
        scale = jnp.broadcast_to(
            (1.0 + jnp.stack([cp["eps"] for cp in cps]))[:, None], (4, _D))
        b1 = jnp.stack([cp["nn"]["l1"]["b"] for cp in cps])
        W2 = jnp.stack([cp["nn"]["l2"]["W"] for cp in cps])
        b2 = jnp.stack([cp["nn"]["l2"]["b"] for cp in cps])
        m = p["mlp_%d" % L]

        src = src_l1 if L == 1 else src_rest
        A = _sc_segsum4(Pflat, src, dst_all, zeros2d)
        R, st = _tc_post(P, A, scale, b1, W2, b2,
                         m["l1"]["W"], m["l1"]["b"].reshape(1, _D),
                         m["l2"]["W"], m["l2"]["b"].reshape(1, _D))
        bn = p["bn%d" % L]
        if L < 4:
            Wn = jnp.stack(
                [p["conv%d_%s" % (L + 1, t)]["nn"]["l1"]["W"] for t in _CONVS])
            h, P = _tc_bn_next(R, st, bn["gamma"].reshape(1, _D),
                               bn["beta"].reshape(1, _D), Wn)
            Pflat = P.reshape(4 * _N, _D)
        else:
            h = _tc_bn_only(R, st, bn["gamma"].reshape(1, _D),
                            bn["beta"].reshape(1, _D))
        outs.append(h)

    batch_r = batch.reshape(_PNB, 1, _PB)
    w4pad = jnp.zeros((_D, _D), jnp.float32).at[:, :1].set(p["fc4"]["W"])
    b4pad = jnp.zeros((1, _D), jnp.float32).at[0, 0].set(p["fc4"]["b"][0])
    z = _tc_pool_fc(outs[0], outs[1], outs[2], outs[3], batch_r,
                    p["fc1"]["W"], p["fc1"]["b"].reshape(1, _D),
                    p["fc2"]["W"], p["fc2"]["b"].reshape(1, _D),
                    p["fc3"]["W"], p["fc3"]["b"].reshape(1, _D),
                    w4pad, b4pad)
    return z[:, 0]


# speedup vs baseline: 4.0043x; 1.5535x over previous
import functools

import jax
import jax.numpy as jnp
from jax import lax
from jax.experimental import pallas as pl
from jax.experimental.pallas import tpu as pltpu
from jax.experimental.pallas import tpu_sc as plsc

_N = 10000
_E = 160000
_F = 652
_D = 128
_G = 64

_NC = 2
_NS = 16
_NW = _NC * _NS

_CH = 64
_HLF = 40
_NSTG = 4
_CPT = _NSTG * _HLF
_EPAD = _NS * _CPT * _CH
_FCH = 128
_AROWS = 10240
_DUMP = _N
_RPT = _AROWS // _NS
_ZCH = _RPT // _FCH

_BN = 400
_NB = _N // _BN
_PB = 200
_PNB = _N // _PB

_mesh = plsc.VectorSubcoreMesh(core_axis_name="c", subcore_axis_name="s")


@functools.partial(
    pl.kernel,
    out_type=jax.ShapeDtypeStruct((4 * _AROWS, _D), jnp.float32),
    mesh=_mesh,
    scratch_types=[
        pltpu.VMEM((10, _FCH), jnp.int32),
        pltpu.VMEM((_FCH, _D), jnp.float32),
        pltpu.SemaphoreType.DMA,
    ],
)
def _sc_gather_p1(tab_hbm, idx_hbm, out_hbm, idx_v, rows_v, sem):
    gwid = lax.axis_index("c") * _NS + lax.axis_index("s")
    pltpu.sync_copy(idx_hbm.at[gwid], idx_v)
    for j in range(10):
        pltpu.async_copy(tab_hbm.at[idx_v.at[j]], rows_v, sem).wait()
        pltpu.sync_copy(rows_v, out_hbm.at[pl.ds(gwid * 1280 + j * _FCH, _FCH)])


@functools.partial(
    pl.kernel,
    out_type=jax.ShapeDtypeStruct((4, _AROWS, _D), jnp.float32),
    mesh=_mesh,
    scratch_types=[
        pltpu.VMEM((_HLF, _CH), jnp.int32),
        pltpu.VMEM((_HLF, _CH), jnp.int32),
        [pltpu.VMEM((_CH, _D), jnp.float32)] * 4,
        pltpu.VMEM_SHARED((_AROWS, _D), jnp.float32),
        [pltpu.SemaphoreType.DMA] * 4,
        [pltpu.SemaphoreType.DMA] * 4,
    ],
)
def _sc_segsum4(p_hbm, src_hbm, dst_hbm, zero_hbm, out_hbm,
                src_v, dst_v, bufs, accum, gsems, ssems):
    cid = lax.axis_index("c")
    sid = lax.axis_index("s")
    for k in range(2):
        conv = 2 * cid + k
        for z in range(_ZCH):
            pltpu.sync_copy(zero_hbm, accum.at[pl.ds(sid * _RPT + z * _FCH, _FCH)])
        plsc.subcore_barrier()
        for half in range(_NSTG):
            pltpu.sync_copy(src_hbm.at[conv, sid, half], src_v)
            pltpu.sync_copy(dst_hbm.at[conv, sid, half], dst_v)
            for i in range(4):
                pltpu.async_copy(p_hbm.at[src_v.at[i]], bufs[i], gsems[i])

            def _body(m, carry):
                j = 4 * m
                jn = lax.min(j + 4, _HLF - 4)
                for i in range(4):
                    pltpu.make_async_copy(p_hbm.at[src_v.at[0]], bufs[i], gsems[i]).wait()
                    pltpu.async_copy(bufs[i], accum.at[dst_v.at[j + i]], ssems[i], add=True)
                    pltpu.make_async_copy(bufs[i], accum.at[dst_v.at[0]], ssems[i]).wait()
                    pltpu.async_copy(p_hbm.at[src_v.at[jn + i]], bufs[i], gsems[i])
                return carry

            lax.fori_loop(0, _HLF // 4, _body, 0)
            for i in range(4):
                pltpu.make_async_copy(p_hbm.at[src_v.at[0]], bufs[i], gsems[i]).wait()
        plsc.subcore_barrier()
        for z in range(_ZCH):
            r0 = sid * _RPT + z * _FCH
            pltpu.sync_copy(accum.at[pl.ds(r0, _FCH)],
                            out_hbm.at[conv, pl.ds(r0, _FCH)])
        plsc.subcore_barrier()


def _post_kernel(p_ref, a_ref, sc_ref, b1_ref, w2_ref, b2_ref,
                 wm1_ref, bm1_ref, wm2_ref, bm2_ref, r_ref, st_ref):
    i = pl.program_id(0)
    xs = []
    for c in range(4):
        u = jnp.maximum(
            sc_ref[c][None, :] * p_ref[c] + a_ref[c] + b1_ref[c][None, :], 0.0)
        xc = jnp.dot(u, w2_ref[c], preferred_element_type=jnp.float32)
        xs.append(jnp.maximum(xc + b2_ref[c][None, :], 0.0))
    cat = jnp.concatenate([xs[0], xs[2], xs[1], xs[3]], axis=1)
    y = jnp.maximum(
        jnp.dot(cat, wm1_ref[...], preferred_element_type=jnp.float32) + bm1_ref[...], 0.0)
    r = jnp.dot(y, wm2_ref[...], preferred_element_type=jnp.float32) + bm2_ref[...]
    r_ref[...] = r
    st = jnp.concatenate(
        [jnp.sum(r, axis=0)[None], jnp.sum(r * r, axis=0)[None],
         jnp.zeros((6, _D), jnp.float32)], axis=0)

    @pl.when(i == 0)
    def _():
        st_ref[...] = st

    @pl.when(i > 0)
    def _():
        st_ref[...] = st_ref[...] + st


def _tc_post(P, A, scale, b1, W2, b2, Wm1, bm1, Wm2, bm2):
    return pl.pallas_call(
        _post_kernel,
        grid=(_NB,),
        in_specs=[
            pl.BlockSpec((4, _BN, _D), lambda i: (0, i, 0)),
            pl.BlockSpec((4, _BN, _D), lambda i: (0, i, 0)),
            pl.BlockSpec((4, _D), lambda i: (0, 0)),
            pl.BlockSpec((4, _D), lambda i: (0, 0)),
            pl.BlockSpec((4, _D, _D), lambda i: (0, 0, 0)),
            pl.BlockSpec((4, _D), lambda i: (0, 0)),
            pl.BlockSpec((4 * _D, _D), lambda i: (0, 0)),
            pl.BlockSpec((1, _D), lambda i: (0, 0)),
            pl.BlockSpec((_D, _D), lambda i: (0, 0)),
            pl.BlockSpec((1, _D), lambda i: (0, 0)),
        ],
        out_specs=[
            pl.BlockSpec((_BN, _D), lambda i: (i, 0)),
            pl.BlockSpec((8, _D), lambda i: (0, 0)),
        ],
        out_shape=[
            jax.ShapeDtypeStruct((_N, _D), jnp.float32),
            jax.ShapeDtypeStruct((8, _D), jnp.float32),
        ],
    )(P, A, scale, b1, W2, b2, Wm1, bm1, Wm2, bm2)


def _bn_next_kernel(r_ref, st_ref, g_ref, bt_ref, wn_ref, h_ref, pn_ref):
    mean = st_ref[0] * (1.0 / _N)
    var = st_ref[1] * (1.0 / _N) - mean * mean
    inv = lax.rsqrt(var + 1e-5) * g_ref[0]
    h = (r_ref[...] - mean[None, :]) * inv[None, :] + bt_ref[...]
    h_ref[...] = h
    for c in range(4):
        pn_ref[c] = jnp.dot(h, wn_ref[c], preferred_element_type=jnp.float32)


def _tc_bn_next(R, st, gamma, beta, Wn):
    return pl.pallas_call(
        _bn_next_kernel,
        grid=(_NB,),
        in_specs=[
            pl.BlockSpec((_BN, _D), lambda i: (i, 0)),
            pl.BlockSpec((8, _D), lambda i: (0, 0)),
            pl.BlockSpec((1, _D), lambda i: (0, 0)),
            pl.BlockSpec((1, _D), lambda i: (0, 0)),
            pl.BlockSpec((4, _D, _D), lambda i: (0, 0, 0)),
        ],
        out_specs=[
            pl.BlockSpec((_BN, _D), lambda i: (i, 0)),
            pl.BlockSpec((4, _BN, _D), lambda i: (0, i, 0)),
        ],
        out_shape=[
            jax.ShapeDtypeStruct((_N, _D), jnp.float32),
            jax.ShapeDtypeStruct((4, _N, _D), jnp.float32),
        ],
    )(R, st, gamma, beta, Wn)


def _bn_only_kernel(r_ref, st_ref, g_ref, bt_ref, h_ref):
    mean = st_ref[0] * (1.0 / _N)
    var = st_ref[1] * (1.0 / _N) - mean * mean
    inv = lax.rsqrt(var + 1e-5) * g_ref[0]
    h_ref[...] = (r_ref[...] - mean[None, :]) * inv[None, :] + bt_ref[...]


def _tc_bn_only(R, st, gamma, beta):
    return pl.pallas_call(
        _bn_only_kernel,
        grid=(_NB,),
        in_specs=[
            pl.BlockSpec((_BN, _D), lambda i: (i, 0)),
            pl.BlockSpec((8, _D), lambda i: (0, 0)),
            pl.BlockSpec((1, _D), lambda i: (0, 0)),
            pl.BlockSpec((1, _D), lambda i: (0, 0)),
        ],
        out_specs=pl.BlockSpec((_BN, _D), lambda i: (i, 0)),
        out_shape=jax.ShapeDtypeStruct((_N, _D), jnp.float32),
    )(R, st, gamma, beta)


def _pool_fc_kernel(h1_ref, h2_ref, h3_ref, h4_ref, b_ref,
                    w1_ref, b1_ref, w2_ref, b2_ref, w3_ref, b3_ref,
                    w4_ref, b4_ref, out_ref, acc_ref, cnt_ref):
    i = pl.program_id(0)

    @pl.when(i == 0)
    def _():
        acc_ref[...] = jnp.zeros_like(acc_ref)
        cnt_ref[...] = jnp.zeros_like(cnt_ref)

    bb = b_ref[...].reshape(_PB)
    gid = lax.broadcasted_iota(jnp.int32, (_G, _PB), 0)
    oh = (bb[None, :] == gid).astype(jnp.float32)
    cat = jnp.concatenate(
        [h1_ref[...], h2_ref[...], h3_ref[...], h4_ref[...]], axis=1)
    acc_ref[...] = acc_ref[...] + jnp.dot(oh, cat, preferred_element_type=jnp.float32)
    cnt_ref[...] = cnt_ref[...] + jnp.broadcast_to(
        jnp.sum(oh, axis=1, keepdims=True), (_G, _D))

    @pl.when(i == _PNB - 1)
    def _():
        cnt = cnt_ref[...][:, :1]
        pooled = acc_ref[...] / jnp.maximum(cnt, 1.0)
        z = jnp.maximum(
            jnp.dot(pooled, w1_ref[...], preferred_element_type=jnp.float32) + b1_ref[...], 0.0)
        z = jnp.maximum(
            jnp.dot(z, w2_ref[...], preferred_element_type=jnp.float32) + b2_ref[...], 0.0)
        z = jnp.maximum(
            jnp.dot(z, w3_ref[...], preferred_element_type=jnp.float32) + b3_ref[...], 0.0)
        out_ref[...] = jnp.dot(z, w4_ref[...], preferred_element_type=jnp.float32) + b4_ref[...]


def _tc_pool_fc(h1, h2, h3, h4, batch_r, w1, b1, w2, b2, w3, b3, w4, b4):
    hspec = pl.BlockSpec((_PB, _D), lambda i: (i, 0))
    return pl.pallas_call(
        _pool_fc_kernel,
        grid=(_PNB,),
        in_specs=[
            hspec, hspec, hspec, hspec,
            pl.BlockSpec((1, 1, _PB), lambda i: (i, 0, 0)),
            pl.BlockSpec((4 * _D, _D), lambda i: (0, 0)),
            pl.BlockSpec((1, _D), lambda i: (0, 0)),
            pl.BlockSpec((_D, _D), lambda i: (0, 0)),
            pl.BlockSpec((1, _D), lambda i: (0, 0)),
            pl.BlockSpec((_D, _D), lambda i: (0, 0)),
            pl.BlockSpec((1, _D), lambda i: (0, 0)),
            pl.BlockSpec((_D, _D), lambda i: (0, 0)),
            pl.BlockSpec((1, _D), lambda i: (0, 0)),
        ],
        out_specs=pl.BlockSpec((_G, _D), lambda i: (0, 0)),
        out_shape=jax.ShapeDtypeStruct((_G, _D), jnp.float32),
        scratch_shapes=[
            pltpu.VMEM((_G, 4 * _D), jnp.float32),
            pltpu.VMEM((_G, _D), jnp.float32),
        ],
    )(h1, h2, h3, h4, batch_r, w1, b1, w2, b2, w3, b3, w4, b4)


_CONVS = ("1_l", "2_l", "1_g", "2_g")


def kernel(x, edge_index_1_l, edge_index_2_l, edge_index_1_g, edge_index_2_g,
           batch, params):
    p = params
    eis = (edge_index_1_l, edge_index_2_l, edge_index_1_g, edge_index_2_g)

    npad = _EPAD - _E
    eshape = (_NS, _NSTG, _HLF, _CH)
    srcs, dsts = [], []
    for ei in eis:
        srcs.append(jnp.concatenate([ei[0], jnp.zeros((npad,), jnp.int32)]))
        dsts.append(jnp.concatenate(
            [ei[1], jnp.full((npad,), _DUMP, jnp.int32)]).reshape(eshape))
    dst_all = jnp.stack(dsts)
    src_l1 = jnp.stack(
        [(s + c * _AROWS).reshape(eshape) for c, s in enumerate(srcs)])
    src_rest = jnp.stack(
        [(s + c * _N).reshape(eshape) for c, s in enumerate(srcs)])
    zeros2d = jnp.zeros((_FCH, _D), jnp.float32)

    l1tab = jnp.concatenate(
        [p["conv1_%s" % t]["nn"]["l1"]["W"] for t in _CONVS], axis=0)
    xpad = jnp.concatenate([x, jnp.zeros((_AROWS - _N,), jnp.int32)])
    xidx = jnp.concatenate([xpad + c * _F for c in range(4)]).reshape(_NW, 10, _FCH)
    Pflat = _sc_gather_p1(l1tab, xidx)
    P = Pflat.reshape(4, _AROWS, _D)

    outs = []
    for L in (1, 2, 3, 4):
        cps = [p["conv%d_%s" % (L, t)] for t in _CONVS]
        scale = jnp.broadcast_to(
            (1.0 + jnp.stack([cp["eps"] for cp in cps]))[:, None], (4, _D))
        b1 = jnp.stack([cp["nn"]["l1"]["b"] for cp in cps])
        W2 = jnp.stack([cp["nn"]["l2"]["W"] for cp in cps])
        b2 = jnp.stack([cp["nn"]["l2"]["b"] for cp in cps])
        m = p["mlp_%d" % L]

        src = src_l1 if L == 1 else src_rest
        A = _sc_segsum4(Pflat, src, dst_all, zeros2d)
        R, st = _tc_post(P, A, scale, b1, W2, b2,
                         m["l1"]["W"], m["l1"]["b"].reshape(1, _D),
                         m["l2"]["W"], m["l2"]["b"].reshape(1, _D))
        bn = p["bn%d" % L]
        if L < 4:
            Wn = jnp.stack(
                [p["conv%d_%s" % (L + 1, t)]["nn"]["l1"]["W"] for t in _CONVS])
            h, P = _tc_bn_next(R, st, bn["gamma"].reshape(1, _D),
                               bn["beta"].reshape(1, _D), Wn)
            Pflat = P.reshape(4 * _N, _D)
        else:
            h = _tc_bn_only(R, st, bn["gamma"].reshape(1, _D),
                            bn["beta"].reshape(1, _D))
        outs.append(h)

    batch_r = batch.reshape(_PNB, 1, _PB)
    w4pad = jnp.zeros((_D, _D), jnp.float32).at[:, :1].set(p["fc4"]["W"])
    b4pad = jnp.zeros((1, _D), jnp.float32).at[0, 0].set(p["fc4"]["b"][0])
    z = _tc_pool_fc(outs[0], outs[1], outs[2], outs[3], batch_r,
                    p["fc1"]["W"], p["fc1"]["b"].reshape(1, _D),
                    p["fc2"]["W"], p["fc2"]["b"].reshape(1, _D),
                    p["fc3"]["W"], p["fc3"]["b"].reshape(1, _D),
                    w4pad, b4pad)
    return z[:, 0]


# speedup vs baseline: 4.1207x; 1.0291x over previous
import functools

import jax
import jax.numpy as jnp
from jax import lax
from jax.experimental import pallas as pl
from jax.experimental.pallas import tpu as pltpu
from jax.experimental.pallas import tpu_sc as plsc

_N = 10000
_E = 160000
_F = 652
_D = 128
_G = 64

_NC = 2
_NS = 16
_NW = _NC * _NS

_CH = 128
_HLF = 40
_NSTG = 2
_CPT = _NSTG * _HLF
_EPAD = _NS * _CPT * _CH
_FCH = 128
_AROWS = 10240
_DUMP = _N
_RPT = _AROWS // _NS
_ZCH = _RPT // _FCH

_BN = 400
_NB = _N // _BN
_PB = 200
_PNB = _N // _PB

_mesh = plsc.VectorSubcoreMesh(core_axis_name="c", subcore_axis_name="s")


@functools.partial(
    pl.kernel,
    out_type=jax.ShapeDtypeStruct((4 * _AROWS, _D), jnp.float32),
    mesh=_mesh,
    scratch_types=[
        pltpu.VMEM((10, _FCH), jnp.int32),
        pltpu.VMEM((_FCH, _D), jnp.float32),
        pltpu.SemaphoreType.DMA,
    ],
)
def _sc_gather_p1(tab_hbm, idx_hbm, out_hbm, idx_v, rows_v, sem):
    gwid = lax.axis_index("c") * _NS + lax.axis_index("s")
    pltpu.sync_copy(idx_hbm.at[gwid], idx_v)
    for j in range(10):
        pltpu.async_copy(tab_hbm.at[idx_v.at[j]], rows_v, sem).wait()
        pltpu.sync_copy(rows_v, out_hbm.at[pl.ds(gwid * 1280 + j * _FCH, _FCH)])


@functools.partial(
    pl.kernel,
    out_type=jax.ShapeDtypeStruct((4, _AROWS, _D), jnp.float32),
    mesh=_mesh,
    scratch_types=[
        pltpu.VMEM((_HLF, _CH), jnp.int32),
        pltpu.VMEM((_HLF, _CH), jnp.int32),
        [pltpu.VMEM((_CH, _D), jnp.float32)] * 2,
        pltpu.VMEM_SHARED((_AROWS, _D), jnp.float32),
        [pltpu.SemaphoreType.DMA] * 2,
        [pltpu.SemaphoreType.DMA] * 2,
    ],
)
def _sc_segsum4(p_hbm, src_hbm, dst_hbm, zero_hbm, out_hbm,
                src_v, dst_v, bufs, accum, gsems, ssems):
    cid = lax.axis_index("c")
    sid = lax.axis_index("s")
    for k in range(2):
        conv = 2 * cid + k
        for z in range(_ZCH):
            pltpu.sync_copy(zero_hbm, accum.at[pl.ds(sid * _RPT + z * _FCH, _FCH)])
        plsc.subcore_barrier()
        for half in range(_NSTG):
            pltpu.sync_copy(src_hbm.at[conv, sid, half], src_v)
            pltpu.sync_copy(dst_hbm.at[conv, sid, half], dst_v)
            for i in range(2):
                pltpu.async_copy(p_hbm.at[src_v.at[i]], bufs[i], gsems[i])

            def _body(m, carry):
                j = 2 * m
                jn = lax.min(j + 2, _HLF - 2)
                for i in range(2):
                    pltpu.make_async_copy(p_hbm.at[src_v.at[0]], bufs[i], gsems[i]).wait()
                    pltpu.async_copy(bufs[i], accum.at[dst_v.at[j + i]], ssems[i], add=True)
                    pltpu.make_async_copy(bufs[i], accum.at[dst_v.at[0]], ssems[i]).wait()
                    pltpu.async_copy(p_hbm.at[src_v.at[jn + i]], bufs[i], gsems[i])
                return carry

            lax.fori_loop(0, _HLF // 2, _body, 0)
            for i in range(2):
                pltpu.make_async_copy(p_hbm.at[src_v.at[0]], bufs[i], gsems[i]).wait()
        plsc.subcore_barrier()
        for z in range(_ZCH):
            r0 = sid * _RPT + z * _FCH
            pltpu.sync_copy(accum.at[pl.ds(r0, _FCH)],
                            out_hbm.at[conv, pl.ds(r0, _FCH)])
        plsc.subcore_barrier()


def _post_kernel(p_ref, a_ref, sc_ref, b1_ref, w2_ref, b2_ref,
                 wm1_ref, bm1_ref, wm2_ref, bm2_ref, r_ref, st_ref):
    i = pl.program_id(0)
    xs = []
    for c in range(4):
        u = jnp.maximum(
            sc_ref[c][None, :] * p_ref[c] + a_ref[c] + b1_ref[c][None, :], 0.0)
        xc = jnp.dot(u, w2_ref[c], preferred_element_type=jnp.float32)
        xs.append(jnp.maximum(xc + b2_ref[c][None, :], 0.0))
    cat = jnp.concatenate([xs[0], xs[2], xs[1], xs[3]], axis=1)
    y = jnp.maximum(
        jnp.dot(cat, wm1_ref[...], preferred_element_type=jnp.float32) + bm1_ref[...], 0.0)
    r = jnp.dot(y, wm2_ref[...], preferred_element_type=jnp.float32) + bm2_ref[...]
    r_ref[...] = r
    st = jnp.concatenate(
        [jnp.sum(r, axis=0)[None], jnp.sum(r * r, axis=0)[None],
         jnp.zeros((6, _D), jnp.float32)], axis=0)

    @pl.when(i == 0)
    def _():
        st_ref[...] = st

    @pl.when(i > 0)
    def _():
        st_ref[...] = st_ref[...] + st


def _tc_post(P, A, scale, b1, W2, b2, Wm1, bm1, Wm2, bm2):
    return pl.pallas_call(
        _post_kernel,
        grid=(_NB,),
        in_specs=[
            pl.BlockSpec((4, _BN, _D), lambda i: (0, i, 0)),
            pl.BlockSpec((4, _BN, _D), lambda i: (0, i, 0)),
            pl.BlockSpec((4, _D), lambda i: (0, 0)),
            pl.BlockSpec((4, _D), lambda i: (0, 0)),
            pl.BlockSpec((4, _D, _D), lambda i: (0, 0, 0)),
            pl.BlockSpec((4, _D), lambda i: (0, 0)),
            pl.BlockSpec((4 * _D, _D), lambda i: (0, 0)),
            pl.BlockSpec((1, _D), lambda i: (0, 0)),
            pl.BlockSpec((_D, _D), lambda i: (0, 0)),
            pl.BlockSpec((1, _D), lambda i: (0, 0)),
        ],
        out_specs=[
            pl.BlockSpec((_BN, _D), lambda i: (i, 0)),
            pl.BlockSpec((8, _D), lambda i: (0, 0)),
        ],
        out_shape=[
            jax.ShapeDtypeStruct((_N, _D), jnp.float32),
            jax.ShapeDtypeStruct((8, _D), jnp.float32),
        ],
    )(P, A, scale, b1, W2, b2, Wm1, bm1, Wm2, bm2)


def _bn_next_kernel(r_ref, st_ref, g_ref, bt_ref, wn_ref, h_ref, pn_ref):
    mean = st_ref[0] * (1.0 / _N)
    var = st_ref[1] * (1.0 / _N) - mean * mean
    inv = lax.rsqrt(var + 1e-5) * g_ref[0]
    h = (r_ref[...] - mean[None, :]) * inv[None, :] + bt_ref[...]
    h_ref[...] = h
    for c in range(4):
        pn_ref[c] = jnp.dot(h, wn_ref[c], preferred_element_type=jnp.float32)


def _tc_bn_next(R, st, gamma, beta, Wn):
    return pl.pallas_call(
        _bn_next_kernel,
        grid=(_NB,),
        in_specs=[
            pl.BlockSpec((_BN, _D), lambda i: (i, 0)),
            pl.BlockSpec((8, _D), lambda i: (0, 0)),
            pl.BlockSpec((1, _D), lambda i: (0, 0)),
            pl.BlockSpec((1, _D), lambda i: (0, 0)),
            pl.BlockSpec((4, _D, _D), lambda i: (0, 0, 0)),
        ],
        out_specs=[
            pl.BlockSpec((_BN, _D), lambda i: (i, 0)),
            pl.BlockSpec((4, _BN, _D), lambda i: (0, i, 0)),
        ],
        out_shape=[
            jax.ShapeDtypeStruct((_N, _D), jnp.float32),
            jax.ShapeDtypeStruct((4, _N, _D), jnp.float32),
        ],
    )(R, st, gamma, beta, Wn)


def _bn_only_kernel(r_ref, st_ref, g_ref, bt_ref, h_ref):
    mean = st_ref[0] * (1.0 / _N)
    var = st_ref[1] * (1.0 / _N) - mean * mean
    inv = lax.rsqrt(var + 1e-5) * g_ref[0]
    h_ref[...] = (r_ref[...] - mean[None, :]) * inv[None, :] + bt_ref[...]


def _tc_bn_only(R, st, gamma, beta):
    return pl.pallas_call(
        _bn_only_kernel,
        grid=(_NB,),
        in_specs=[
            pl.BlockSpec((_BN, _D), lambda i: (i, 0)),
            pl.BlockSpec((8, _D), lambda i: (0, 0)),
            pl.BlockSpec((1, _D), lambda i: (0, 0)),
            pl.BlockSpec((1, _D), lambda i: (0, 0)),
        ],
        out_specs=pl.BlockSpec((_BN, _D), lambda i: (i, 0)),
        out_shape=jax.ShapeDtypeStruct((_N, _D), jnp.float32),
    )(R, st, gamma, beta)


def _pool_fc_kernel(h1_ref, h2_ref, h3_ref, h4_ref, b_ref,
                    w1_ref, b1_ref, w2_ref, b2_ref, w3_ref, b3_ref,
                    w4_ref, b4_ref, out_ref, acc_ref, cnt_ref):
    i = pl.program_id(0)

    @pl.when(i == 0)
    def _():
        acc_ref[...] = jnp.zeros_like(acc_ref)
        cnt_ref[...] = jnp.zeros_like(cnt_ref)

    bb = b_ref[...].reshape(_PB)
    gid = lax.broadcasted_iota(jnp.int32, (_G, _PB), 0)
    oh = (bb[None, :] == gid).astype(jnp.float32)
    cat = jnp.concatenate(
        [h1_ref[...], h2_ref[...], h3_ref[...], h4_ref[...]], axis=1)
    acc_ref[...] = acc_ref[...] + jnp.dot(oh, cat, preferred_element_type=jnp.float32)
    cnt_ref[...] = cnt_ref[...] + jnp.broadcast_to(
        jnp.sum(oh, axis=1, keepdims=True), (_G, _D))

    @pl.when(i == _PNB - 1)
    def _():
        cnt = cnt_ref[...][:, :1]
        pooled = acc_ref[...] / jnp.maximum(cnt, 1.0)
        z = jnp.maximum(
            jnp.dot(pooled, w1_ref[...], preferred_element_type=jnp.float32) + b1_ref[...], 0.0)
        z = jnp.maximum(
            jnp.dot(z, w2_ref[...], preferred_element_type=jnp.float32) + b2_ref[...], 0.0)
        z = jnp.maximum(
            jnp.dot(z, w3_ref[...], preferred_element_type=jnp.float32) + b3_ref[...], 0.0)
        out_ref[...] = jnp.dot(z, w4_ref[...], preferred_element_type=jnp.float32) + b4_ref[...]


def _tc_pool_fc(h1, h2, h3, h4, batch_r, w1, b1, w2, b2, w3, b3, w4, b4):
    hspec = pl.BlockSpec((_PB, _D), lambda i: (i, 0))
    return pl.pallas_call(
        _pool_fc_kernel,
        grid=(_PNB,),
        in_specs=[
            hspec, hspec, hspec, hspec,
            pl.BlockSpec((1, 1, _PB), lambda i: (i, 0, 0)),
            pl.BlockSpec((4 * _D, _D), lambda i: (0, 0)),
            pl.BlockSpec((1, _D), lambda i: (0, 0)),
            pl.BlockSpec((_D, _D), lambda i: (0, 0)),
            pl.BlockSpec((1, _D), lambda i: (0, 0)),
            pl.BlockSpec((_D, _D), lambda i: (0, 0)),
            pl.BlockSpec((1, _D), lambda i: (0, 0)),
            pl.BlockSpec((_D, _D), lambda i: (0, 0)),
            pl.BlockSpec((1, _D), lambda i: (0, 0)),
        ],
        out_specs=pl.BlockSpec((_G, _D), lambda i: (0, 0)),
        out_shape=jax.ShapeDtypeStruct((_G, _D), jnp.float32),
        scratch_shapes=[
            pltpu.VMEM((_G, 4 * _D), jnp.float32),
            pltpu.VMEM((_G, _D), jnp.float32),
        ],
    )(h1, h2, h3, h4, batch_r, w1, b1, w2, b2, w3, b3, w4, b4)


_CONVS = ("1_l", "2_l", "1_g", "2_g")


def kernel(x, edge_index_1_l, edge_index_2_l, edge_index_1_g, edge_index_2_g,
           batch, params):
    p = params
    eis = (edge_index_1_l, edge_index_2_l, edge_index_1_g, edge_index_2_g)

    npad = _EPAD - _E
    eshape = (_NS, _NSTG, _HLF, _CH)
    srcs, dsts = [], []
    for ei in eis:
        srcs.append(jnp.concatenate([ei[0], jnp.zeros((npad,), jnp.int32)]))
        dsts.append(jnp.concatenate(
            [ei[1], jnp.full((npad,), _DUMP, jnp.int32)]).reshape(eshape))
    dst_all = jnp.stack(dsts)
    src_l1 = jnp.stack(
        [(s + c * _AROWS).reshape(eshape) for c, s in enumerate(srcs)])
    src_rest = jnp.stack(
        [(s + c * _N).reshape(eshape) for c, s in enumerate(srcs)])
    zeros2d = jnp.zeros((_FCH, _D), jnp.float32)

    l1tab = jnp.concatenate(
        [p["conv1_%s" % t]["nn"]["l1"]["W"] for t in _CONVS], axis=0)
    xpad = jnp.concatenate([x, jnp.zeros((_AROWS - _N,), jnp.int32)])
    xidx = jnp.concatenate([xpad + c * _F for c in range(4)]).reshape(_NW, 10, _FCH)
    Pflat = _sc_gather_p1(l1tab, xidx)
    P = Pflat.reshape(4, _AROWS, _D)

    outs = []
    for L in (1, 2, 3, 4):
        cps = [p["conv%d_%s" % (L, t)] for t in _CONVS]
        scale = jnp.broadcast_to(
            (1.0 + jnp.stack([cp["eps"] for cp in cps]))[:, None], (4, _D))
        b1 = jnp.stack([cp["nn"]["l1"]["b"] for cp in cps])
        W2 = jnp.stack([cp["nn"]["l2"]["W"] for cp in cps])
        b2 = jnp.stack([cp["nn"]["l2"]["b"] for cp in cps])
        m = p["mlp_%d" % L]

        src = src_l1 if L == 1 else src_rest
        A = _sc_segsum4(Pflat, src, dst_all, zeros2d)
        R, st = _tc_post(P, A, scale, b1, W2, b2,
                         m["l1"]["W"], m["l1"]["b"].reshape(1, _D),
                         m["l2"]["W"], m["l2"]["b"].reshape(1, _D))
        bn = p["bn%d" % L]
        if L < 4:
            Wn = jnp.stack(
                [p["conv%d_%s" % (L + 1, t)]["nn"]["l1"]["W"] for t in _CONVS])
            h, P = _tc_bn_next(R, st, bn["gamma"].reshape(1, _D),
                               bn["beta"].reshape(1, _D), Wn)
            Pflat = P.reshape(4 * _N, _D)
        else:
            h = _tc_bn_only(R, st, bn["gamma"].reshape(1, _D),
                            bn["beta"].reshape(1, _D))
        outs.append(h)

    batch_r = batch.reshape(_PNB, 1, _PB)
    w4pad = jnp.zeros((_D, _D), jnp.float32).at[:, :1].set(p["fc4"]["W"])
    b4pad = jnp.zeros((1, _D), jnp.float32).at[0, 0].set(p["fc4"]["b"][0])
    z = _tc_pool_fc(outs[0], outs[1], outs[2], outs[3], batch_r,
                    p["fc1"]["W"], p["fc1"]["b"].reshape(1, _D),
                    p["fc2"]["W"], p["fc2"]["b"].reshape(1, _D),
                    p["fc3"]["W"], p["fc3"]["b"].reshape(1, _D),
                    w4pad, b4pad)
    return z[:, 0]


# speedup vs baseline: 4.4478x; 1.0794x over previous
import functools

import jax
import jax.numpy as jnp
from jax import lax
from jax.experimental import pallas as pl
from jax.experimental.pallas import tpu as pltpu
from jax.experimental.pallas import tpu_sc as plsc

_N = 10000
_E = 160000
_F = 652
_D = 128
_G = 64

_NC = 2
_NS = 16
_NW = _NC * _NS

_CH = 128
_HLF = 40
_NSTG = 2
_CPT = _NSTG * _HLF
_EPAD = _NS * _CPT * _CH
_FCH = 128
_AROWS = 10240
_DUMP = _N
_RPT = _AROWS // _NS
_ZCH = _RPT // _FCH

_BN = 400
_NB = _N // _BN
_PB = 200
_PNB = _N // _PB

_mesh = plsc.VectorSubcoreMesh(core_axis_name="c", subcore_axis_name="s")


@functools.partial(
    pl.kernel,
    out_type=jax.ShapeDtypeStruct((4 * _AROWS, _D), jnp.float32),
    mesh=_mesh,
    scratch_types=[
        pltpu.VMEM((10, _FCH), jnp.int32),
        pltpu.VMEM((_FCH, _D), jnp.float32),
        pltpu.SemaphoreType.DMA,
    ],
)
def _sc_gather_p1(tab_hbm, idx_hbm, out_hbm, idx_v, rows_v, sem):
    gwid = lax.axis_index("c") * _NS + lax.axis_index("s")
    pltpu.sync_copy(idx_hbm.at[gwid], idx_v)
    for j in range(10):
        pltpu.async_copy(tab_hbm.at[idx_v.at[j]], rows_v, sem).wait()
        pltpu.sync_copy(rows_v, out_hbm.at[pl.ds(gwid * 1280 + j * _FCH, _FCH)])


@functools.partial(
    pl.kernel,
    out_type=jax.ShapeDtypeStruct((4, _AROWS, _D), jnp.float32),
    mesh=_mesh,
    scratch_types=[
        pltpu.VMEM((_HLF, _CH), jnp.int32),
        pltpu.VMEM((_HLF, _CH), jnp.int32),
        [pltpu.VMEM((_CH, _D), jnp.float32)] * 2,
        pltpu.VMEM_SHARED((_AROWS, _D), jnp.float32),
        [pltpu.SemaphoreType.DMA] * 2,
        [pltpu.SemaphoreType.DMA] * 2,
    ],
)
def _sc_segsum4(p_hbm, src_hbm, dst_hbm, zero_hbm, out_hbm,
                src_v, dst_v, bufs, accum, gsems, ssems):
    cid = lax.axis_index("c")
    sid = lax.axis_index("s")
    for k in range(2):
        conv = 2 * cid + k
        for z in range(_ZCH):
            pltpu.sync_copy(zero_hbm, accum.at[pl.ds(sid * _RPT + z * _FCH, _FCH)])
        plsc.subcore_barrier()
        for half in range(_NSTG):
            pltpu.sync_copy(src_hbm.at[conv, sid, half], src_v)
            pltpu.sync_copy(dst_hbm.at[conv, sid, half], dst_v)
            for i in range(2):
                pltpu.async_copy(p_hbm.at[src_v.at[i]], bufs[i], gsems[i])

            def _body(m, carry):
                j = 2 * m
                jn = lax.min(j + 2, _HLF - 2)
                for i in range(2):
                    pltpu.make_async_copy(p_hbm.at[src_v.at[0]], bufs[i], gsems[i]).wait()
                    pltpu.async_copy(bufs[i], accum.at[dst_v.at[j + i]], ssems[i], add=True)
                    pltpu.make_async_copy(bufs[i], accum.at[dst_v.at[0]], ssems[i]).wait()
                    pltpu.async_copy(p_hbm.at[src_v.at[jn + i]], bufs[i], gsems[i])
                return carry

            lax.fori_loop(0, _HLF // 2, _body, 0)
            for i in range(2):
                pltpu.make_async_copy(p_hbm.at[src_v.at[0]], bufs[i], gsems[i]).wait()
        plsc.subcore_barrier()
        for z in range(_ZCH):
            r0 = sid * _RPT + z * _FCH
            pltpu.sync_copy(accum.at[pl.ds(r0, _FCH)],
                            out_hbm.at[conv, pl.ds(r0, _FCH)])
        plsc.subcore_barrier()


def _make_post_bn(has_next):
    def body(p_ref, a_ref, sc_ref, b1_ref, w2_ref, b2_ref,
             wm1_ref, bm1_ref, wm2_ref, bm2_ref, g_ref, bt_ref, *rest):
        if has_next:
            wn_ref, h_ref, pn_ref, r_scr, st_scr = rest
        else:
            h_ref, r_scr, st_scr = rest
        ph = pl.program_id(0)
        i = pl.program_id(1)

        @pl.when(ph == 0)
        def _():
            xs = []
            for c in range(4):
                u = jnp.maximum(
                    sc_ref[c][None, :] * p_ref[c] + a_ref[c] + b1_ref[c][None, :], 0.0)
                xc = jnp.dot(u, w2_ref[c], preferred_element_type=jnp.float32)
                xs.append(jnp.maximum(xc + b2_ref[c][None, :], 0.0))
            cat = jnp.concatenate([xs[0], xs[2], xs[1], xs[3]], axis=1)
            y = jnp.maximum(
                jnp.dot(cat, wm1_ref[...], preferred_element_type=jnp.float32)
                + bm1_ref[...], 0.0)
            r = jnp.dot(y, wm2_ref[...], preferred_element_type=jnp.float32) + bm2_ref[...]
            r_scr[pl.ds(i * _BN, _BN), :] = r
            st = jnp.concatenate(
                [jnp.sum(r, axis=0)[None], jnp.sum(r * r, axis=0)[None],
                 jnp.zeros((6, _D), jnp.float32)], axis=0)

            @pl.when(i == 0)
            def _():
                st_scr[...] = st

            @pl.when(i > 0)
            def _():
                st_scr[...] = st_scr[...] + st

        @pl.when(ph == 1)
        def _():
            mean = st_scr[0] * (1.0 / _N)
            var = st_scr[1] * (1.0 / _N) - mean * mean
            inv = lax.rsqrt(var + 1e-5) * g_ref[0]
            r = r_scr[pl.ds(i * _BN, _BN), :]
            h = (r - mean[None, :]) * inv[None, :] + bt_ref[...]
            h_ref[...] = h
            if has_next:
                for c in range(4):
                    pn_ref[c] = jnp.dot(h, wn_ref[c], preferred_element_type=jnp.float32)

    in_specs = [
        pl.BlockSpec((4, _BN, _D), lambda p, i: (0, jnp.where(p == 0, i, _NB - 1), 0)),
        pl.BlockSpec((4, _BN, _D), lambda p, i: (0, jnp.where(p == 0, i, _NB - 1), 0)),
        pl.BlockSpec((4, _D), lambda p, i: (0, 0)),
        pl.BlockSpec((4, _D), lambda p, i: (0, 0)),
        pl.BlockSpec((4, _D, _D), lambda p, i: (0, 0, 0)),
        pl.BlockSpec((4, _D), lambda p, i: (0, 0)),
        pl.BlockSpec((4 * _D, _D), lambda p, i: (0, 0)),
        pl.BlockSpec((1, _D), lambda p, i: (0, 0)),
        pl.BlockSpec((_D, _D), lambda p, i: (0, 0)),
        pl.BlockSpec((1, _D), lambda p, i: (0, 0)),
        pl.BlockSpec((1, _D), lambda p, i: (0, 0)),
        pl.BlockSpec((1, _D), lambda p, i: (0, 0)),
    ]
    out_specs = [pl.BlockSpec((_BN, _D), lambda p, i: (jnp.where(p == 1, i, 0), 0))]
    out_shape = [jax.ShapeDtypeStruct((_N, _D), jnp.float32)]
    if has_next:
        in_specs.append(pl.BlockSpec((4, _D, _D), lambda p, i: (0, 0, 0)))
        out_specs.append(
            pl.BlockSpec((4, _BN, _D), lambda p, i: (0, jnp.where(p == 1, i, 0), 0)))
        out_shape.append(jax.ShapeDtypeStruct((4, _AROWS, _D), jnp.float32))

    def call(*args):
        return pl.pallas_call(
            body,
            grid=(2, _NB),
            in_specs=in_specs,
            out_specs=out_specs,
            out_shape=out_shape,
            scratch_shapes=[
                pltpu.VMEM((_N, _D), jnp.float32),
                pltpu.VMEM((8, _D), jnp.float32),
            ],
        )(*args)

    return call


_tc_post_bn_next = _make_post_bn(True)
_tc_post_bn_last = _make_post_bn(False)


def _pool_fc_kernel(h1_ref, h2_ref, h3_ref, h4_ref, b_ref,
                    w1_ref, b1_ref, w2_ref, b2_ref, w3_ref, b3_ref,
                    w4_ref, b4_ref, out_ref, acc_ref, cnt_ref):
    i = pl.program_id(0)

    @pl.when(i == 0)
    def _():
        acc_ref[...] = jnp.zeros_like(acc_ref)
        cnt_ref[...] = jnp.zeros_like(cnt_ref)

    bb = b_ref[...].reshape(_PB)
    gid = lax.broadcasted_iota(jnp.int32, (_G, _PB), 0)
    oh = (bb[None, :] == gid).astype(jnp.float32)
    cat = jnp.concatenate(
        [h1_ref[...], h2_ref[...], h3_ref[...], h4_ref[...]], axis=1)
    acc_ref[...] = acc_ref[...] + jnp.dot(oh, cat, preferred_element_type=jnp.float32)
    cnt_ref[...] = cnt_ref[...] + jnp.broadcast_to(
        jnp.sum(oh, axis=1, keepdims=True), (_G, _D))

    @pl.when(i == _PNB - 1)
    def _():
        cnt = cnt_ref[...][:, :1]
        pooled = acc_ref[...] / jnp.maximum(cnt, 1.0)
        z = jnp.maximum(
            jnp.dot(pooled, w1_ref[...], preferred_element_type=jnp.float32) + b1_ref[...], 0.0)
        z = jnp.maximum(
            jnp.dot(z, w2_ref[...], preferred_element_type=jnp.float32) + b2_ref[...], 0.0)
        z = jnp.maximum(
            jnp.dot(z, w3_ref[...], preferred_element_type=jnp.float32) + b3_ref[...], 0.0)
        out_ref[...] = jnp.dot(z, w4_ref[...], preferred_element_type=jnp.float32) + b4_ref[...]


def _tc_pool_fc(h1, h2, h3, h4, batch_r, w1, b1, w2, b2, w3, b3, w4, b4):
    hspec = pl.BlockSpec((_PB, _D), lambda i: (i, 0))
    return pl.pallas_call(
        _pool_fc_kernel,
        grid=(_PNB,),
        in_specs=[
            hspec, hspec, hspec, hspec,
            pl.BlockSpec((1, 1, _PB), lambda i: (i, 0, 0)),
            pl.BlockSpec((4 * _D, _D), lambda i: (0, 0)),
            pl.BlockSpec((1, _D), lambda i: (0, 0)),
            pl.BlockSpec((_D, _D), lambda i: (0, 0)),
            pl.BlockSpec((1, _D), lambda i: (0, 0)),
            pl.BlockSpec((_D, _D), lambda i: (0, 0)),
            pl.BlockSpec((1, _D), lambda i: (0, 0)),
            pl.BlockSpec((_D, _D), lambda i: (0, 0)),
            pl.BlockSpec((1, _D), lambda i: (0, 0)),
        ],
        out_specs=pl.BlockSpec((_G, _D), lambda i: (0, 0)),
        out_shape=jax.ShapeDtypeStruct((_G, _D), jnp.float32),
        scratch_shapes=[
            pltpu.VMEM((_G, 4 * _D), jnp.float32),
            pltpu.VMEM((_G, _D), jnp.float32),
        ],
    )(h1, h2, h3, h4, batch_r, w1, b1, w2, b2, w3, b3, w4, b4)


_CONVS = ("1_l", "2_l", "1_g", "2_g")


def kernel(x, edge_index_1_l, edge_index_2_l, edge_index_1_g, edge_index_2_g,
           batch, params):
    p = params
    eis = (edge_index_1_l, edge_index_2_l, edge_index_1_g, edge_index_2_g)

    npad = _EPAD - _E
    eshape = (_NS, _NSTG, _HLF, _CH)
    srcs, dsts = [], []
    for ei in eis:
        srcs.append(jnp.concatenate([ei[0], jnp.zeros((npad,), jnp.int32)]))
        dsts.append(jnp.concatenate(
            [ei[1], jnp.full((npad,), _DUMP, jnp.int32)]).reshape(eshape))
    dst_all = jnp.stack(dsts)
    src_all = jnp.stack(
        [(s + c * _AROWS).reshape(eshape) for c, s in enumerate(srcs)])
    zeros2d = jnp.zeros((_FCH, _D), jnp.float32)

    l1tab = jnp.concatenate(
        [p["conv1_%s" % t]["nn"]["l1"]["W"] for t in _CONVS], axis=0)
    xpad = jnp.concatenate([x, jnp.zeros((_AROWS - _N,), jnp.int32)])
    xidx = jnp.concatenate([xpad + c * _F for c in range(4)]).reshape(_NW, 10, _FCH)
    Pflat = _sc_gather_p1(l1tab, xidx)
    P = Pflat.reshape(4, _AROWS, _D)

    outs = []
    for L in (1, 2, 3, 4):
        cps = [p["conv%d_%s" % (L, t)] for t in _CONVS]
        scale = jnp.broadcast_to(
            (1.0 + jnp.stack([cp["eps"] for cp in cps]))[:, None], (4, _D))
        b1 = jnp.stack([cp["nn"]["l1"]["b"] for cp in cps])
        W2 = jnp.stack([cp["nn"]["l2"]["W"] for cp in cps])
        b2 = jnp.stack([cp["nn"]["l2"]["b"] for cp in cps])
        m = p["mlp_%d" % L]

        A = _sc_segsum4(Pflat, src_all, dst_all, zeros2d)
        bn = p["bn%d" % L]
        common = (P, A, scale, b1, W2, b2,
                  m["l1"]["W"], m["l1"]["b"].reshape(1, _D),
                  m["l2"]["W"], m["l2"]["b"].reshape(1, _D),
                  bn["gamma"].reshape(1, _D), bn["beta"].reshape(1, _D))
        if L < 4:
            Wn = jnp.stack(
                [p["conv%d_%s" % (L + 1, t)]["nn"]["l1"]["W"] for t in _CONVS])
            h, P = _tc_post_bn_next(*common, Wn)
            Pflat = P.reshape(4 * _AROWS, _D)
        else:
            (h,) = _tc_post_bn_last(*common)
        outs.append(h)

    batch_r = batch.reshape(_PNB, 1, _PB)
    w4pad = jnp.zeros((_D, _D), jnp.float32).at[:, :1].set(p["fc4"]["W"])
    b4pad = jnp.zeros((1, _D), jnp.float32).at[0, 0].set(p["fc4"]["b"][0])
    z = _tc_pool_fc(outs[0], outs[1], outs[2], outs[3], batch_r,
                    p["fc1"]["W"], p["fc1"]["b"].reshape(1, _D),
                    p["fc2"]["W"], p["fc2"]["b"].reshape(1, _D),
                    p["fc3"]["W"], p["fc3"]["b"].reshape(1, _D),
                    w4pad, b4pad)
    return z[:, 0]


# speedup vs baseline: 4.6256x; 1.0400x over previous
import functools

import jax
import jax.numpy as jnp
from jax import lax
from jax.experimental import pallas as pl
from jax.experimental.pallas import tpu as pltpu
from jax.experimental.pallas import tpu_sc as plsc

_N = 10000
_E = 160000
_F = 652
_D = 128
_G = 64

_NC = 2
_NS = 16
_NW = _NC * _NS

_CH = 128
_HLF = 40
_NSTG = 2
_CPT = _NSTG * _HLF
_EPAD = _NS * _CPT * _CH
_FCH = 128
_AROWS = 10240
_DUMP = _N
_RPT = _AROWS // _NS
_ZCH = _RPT // _FCH

_BN = 2000
_NB = _N // _BN
_PB = 1000
_PNB = _N // _PB

_mesh = plsc.VectorSubcoreMesh(core_axis_name="c", subcore_axis_name="s")


@functools.partial(
    pl.kernel,
    out_type=jax.ShapeDtypeStruct((4 * _AROWS, _D), jnp.float32),
    mesh=_mesh,
    scratch_types=[
        pltpu.VMEM((10, _FCH), jnp.int32),
        [pltpu.VMEM((_FCH, _D), jnp.float32)] * 2,
        [pltpu.SemaphoreType.DMA] * 2,
        [pltpu.SemaphoreType.DMA] * 2,
    ],
)
def _sc_gather_p1(tab_hbm, idx_hbm, out_hbm, idx_v, rows, gsems, wsems):
    gwid = lax.axis_index("c") * _NS + lax.axis_index("s")
    pltpu.sync_copy(idx_hbm.at[gwid], idx_v)
    for i in range(2):
        pltpu.async_copy(tab_hbm.at[idx_v.at[i]], rows[i], gsems[i])
    for j in range(10):
        i = j % 2
        pltpu.make_async_copy(tab_hbm.at[idx_v.at[0]], rows[i], gsems[i]).wait()
        pltpu.async_copy(rows[i], out_hbm.at[pl.ds(gwid * 1280 + j * _FCH, _FCH)],
                         wsems[i])
        if j < 8:
            pltpu.make_async_copy(
                rows[i], out_hbm.at[pl.ds(0, _FCH)], wsems[i]).wait()
            pltpu.async_copy(tab_hbm.at[idx_v.at[j + 2]], rows[i], gsems[i])
    for i in range(2):
        pltpu.make_async_copy(rows[i], out_hbm.at[pl.ds(0, _FCH)], wsems[i]).wait()


@functools.partial(
    pl.kernel,
    out_type=jax.ShapeDtypeStruct((4, _AROWS, _D), jnp.float32),
    mesh=_mesh,
    scratch_types=[
        pltpu.VMEM((_HLF, _CH), jnp.int32),
        pltpu.VMEM((_HLF, _CH), jnp.int32),
        [pltpu.VMEM((_CH, _D), jnp.float32)] * 2,
        pltpu.VMEM_SHARED((_AROWS, _D), jnp.float32),
        [pltpu.SemaphoreType.DMA] * 2,
        [pltpu.SemaphoreType.DMA] * 2,
    ],
)
def _sc_segsum4(p_hbm, src_hbm, dst_hbm, zero_hbm, out_hbm,
                src_v, dst_v, bufs, accum, gsems, ssems):
    cid = lax.axis_index("c")
    sid = lax.axis_index("s")
    for k in range(2):
        conv = 2 * cid + k
        for z in range(_ZCH):
            pltpu.sync_copy(zero_hbm, accum.at[pl.ds(sid * _RPT + z * _FCH, _FCH)])
        plsc.subcore_barrier()
        for half in range(_NSTG):
            pltpu.sync_copy(src_hbm.at[conv, sid, half], src_v)
            pltpu.sync_copy(dst_hbm.at[conv, sid, half], dst_v)
            for i in range(2):
                pltpu.async_copy(p_hbm.at[src_v.at[i]], bufs[i], gsems[i])

            def _body(m, carry):
                j = 2 * m
                jn = lax.min(j + 2, _HLF - 2)
                for i in range(2):
                    pltpu.make_async_copy(p_hbm.at[src_v.at[0]], bufs[i], gsems[i]).wait()
                    pltpu.async_copy(bufs[i], accum.at[dst_v.at[j + i]], ssems[i], add=True)
                    pltpu.make_async_copy(bufs[i], accum.at[dst_v.at[0]], ssems[i]).wait()
                    pltpu.async_copy(p_hbm.at[src_v.at[jn + i]], bufs[i], gsems[i])
                return carry

            lax.fori_loop(0, _HLF // 2, _body, 0)
            for i in range(2):
                pltpu.make_async_copy(p_hbm.at[src_v.at[0]], bufs[i], gsems[i]).wait()
        plsc.subcore_barrier()
        for z in range(_ZCH):
            r0 = sid * _RPT + z * _FCH
            pltpu.sync_copy(accum.at[pl.ds(r0, _FCH)],
                            out_hbm.at[conv, pl.ds(r0, _FCH)])
        plsc.subcore_barrier()


def _make_post_bn(has_next):
    def body(p_ref, a_ref, sc_ref, b1_ref, w2_ref, b2_ref,
             wm1_ref, bm1_ref, wm2_ref, bm2_ref, g_ref, bt_ref, *rest):
        if has_next:
            wn_ref, h_ref, pn_ref, r_scr, st_scr = rest
        else:
            h_ref, r_scr, st_scr = rest
        ph = pl.program_id(0)
        i = pl.program_id(1)

        @pl.when(ph == 0)
        def _():
            xs = []
            for c in range(4):
                u = jnp.maximum(
                    sc_ref[c][None, :] * p_ref[c] + a_ref[c] + b1_ref[c][None, :], 0.0)
                xc = jnp.dot(u, w2_ref[c], preferred_element_type=jnp.float32)
                xs.append(jnp.maximum(xc + b2_ref[c][None, :], 0.0))
            cat = jnp.concatenate([xs[0], xs[2], xs[1], xs[3]], axis=1)
            y = jnp.maximum(
                jnp.dot(cat, wm1_ref[...], preferred_element_type=jnp.float32)
                + bm1_ref[...], 0.0)
            r = jnp.dot(y, wm2_ref[...], preferred_element_type=jnp.float32) + bm2_ref[...]
            r_scr[pl.ds(i * _BN, _BN), :] = r
            st = jnp.concatenate(
                [jnp.sum(r, axis=0)[None], jnp.sum(r * r, axis=0)[None],
                 jnp.zeros((6, _D), jnp.float32)], axis=0)

            @pl.when(i == 0)
            def _():
                st_scr[...] = st

            @pl.when(i > 0)
            def _():
                st_scr[...] = st_scr[...] + st

        @pl.when(ph == 1)
        def _():
            mean = st_scr[0] * (1.0 / _N)
            var = st_scr[1] * (1.0 / _N) - mean * mean
            inv = lax.rsqrt(var + 1e-5) * g_ref[0]
            r = r_scr[pl.ds(i * _BN, _BN), :]
            h = (r - mean[None, :]) * inv[None, :] + bt_ref[...]
            h_ref[...] = h
            if has_next:
                for c in range(4):
                    pn_ref[c] = jnp.dot(h, wn_ref[c], preferred_element_type=jnp.float32)

    in_specs = [
        pl.BlockSpec((4, _BN, _D), lambda p, i: (0, jnp.where(p == 0, i, _NB - 1), 0)),
        pl.BlockSpec((4, _BN, _D), lambda p, i: (0, jnp.where(p == 0, i, _NB - 1), 0)),
        pl.BlockSpec((4, _D), lambda p, i: (0, 0)),
        pl.BlockSpec((4, _D), lambda p, i: (0, 0)),
        pl.BlockSpec((4, _D, _D), lambda p, i: (0, 0, 0)),
        pl.BlockSpec((4, _D), lambda p, i: (0, 0)),
        pl.BlockSpec((4 * _D, _D), lambda p, i: (0, 0)),
        pl.BlockSpec((1, _D), lambda p, i: (0, 0)),
        pl.BlockSpec((_D, _D), lambda p, i: (0, 0)),
        pl.BlockSpec((1, _D), lambda p, i: (0, 0)),
        pl.BlockSpec((1, _D), lambda p, i: (0, 0)),
        pl.BlockSpec((1, _D), lambda p, i: (0, 0)),
    ]
    out_specs = [pl.BlockSpec((_BN, _D), lambda p, i: (jnp.where(p == 1, i, 0), 0))]
    out_shape = [jax.ShapeDtypeStruct((_N, _D), jnp.float32)]
    if has_next:
        in_specs.append(pl.BlockSpec((4, _D, _D), lambda p, i: (0, 0, 0)))
        out_specs.append(
            pl.BlockSpec((4, _BN, _D), lambda p, i: (0, jnp.where(p == 1, i, 0), 0)))
        out_shape.append(jax.ShapeDtypeStruct((4, _AROWS, _D), jnp.float32))

    def call(*args):
        return pl.pallas_call(
            body,
            grid=(2, _NB),
            in_specs=in_specs,
            out_specs=out_specs,
            out_shape=out_shape,
            scratch_shapes=[
                pltpu.VMEM((_N, _D), jnp.float32),
                pltpu.VMEM((8, _D), jnp.float32),
            ],
        )(*args)

    return call


_tc_post_bn_next = _make_post_bn(True)
_tc_post_bn_last = _make_post_bn(False)


def _pool_fc_kernel(h1_ref, h2_ref, h3_ref, h4_ref, b_ref,
                    w1_ref, b1_ref, w2_ref, b2_ref, w3_ref, b3_ref,
                    w4_ref, b4_ref, out_ref, acc_ref, cnt_ref):
    i = pl.program_id(0)

    @pl.when(i == 0)
    def _():
        acc_ref[...] = jnp.zeros_like(acc_ref)
        cnt_ref[...] = jnp.zeros_like(cnt_ref)

    bb = b_ref[...].reshape(_PB)
    gid = lax.broadcasted_iota(jnp.int32, (_G, _PB), 0)
    oh = (bb[None, :] == gid).astype(jnp.float32)
    cat = jnp.concatenate(
        [h1_ref[...], h2_ref[...], h3_ref[...], h4_ref[...]], axis=1)
    acc_ref[...] = acc_ref[...] + jnp.dot(oh, cat, preferred_element_type=jnp.float32)
    cnt_ref[...] = cnt_ref[...] + jnp.broadcast_to(
        jnp.sum(oh, axis=1, keepdims=True), (_G, _D))

    @pl.when(i == _PNB - 1)
    def _():
        cnt = cnt_ref[...][:, :1]
        pooled = acc_ref[...] / jnp.maximum(cnt, 1.0)
        z = jnp.maximum(
            jnp.dot(pooled, w1_ref[...], preferred_element_type=jnp.float32) + b1_ref[...], 0.0)
        z = jnp.maximum(
            jnp.dot(z, w2_ref[...], preferred_element_type=jnp.float32) + b2_ref[...], 0.0)
        z = jnp.maximum(
            jnp.dot(z, w3_ref[...], preferred_element_type=jnp.float32) + b3_ref[...], 0.0)
        out_ref[...] = jnp.dot(z, w4_ref[...], preferred_element_type=jnp.float32) + b4_ref[...]


def _tc_pool_fc(h1, h2, h3, h4, batch_r, w1, b1, w2, b2, w3, b3, w4, b4):
    hspec = pl.BlockSpec((_PB, _D), lambda i: (i, 0))
    return pl.pallas_call(
        _pool_fc_kernel,
        grid=(_PNB,),
        in_specs=[
            hspec, hspec, hspec, hspec,
            pl.BlockSpec((1, 1, _PB), lambda i: (i, 0, 0)),
            pl.BlockSpec((4 * _D, _D), lambda i: (0, 0)),
            pl.BlockSpec((1, _D), lambda i: (0, 0)),
            pl.BlockSpec((_D, _D), lambda i: (0, 0)),
            pl.BlockSpec((1, _D), lambda i: (0, 0)),
            pl.BlockSpec((_D, _D), lambda i: (0, 0)),
            pl.BlockSpec((1, _D), lambda i: (0, 0)),
            pl.BlockSpec((_D, _D), lambda i: (0, 0)),
            pl.BlockSpec((1, _D), lambda i: (0, 0)),
        ],
        out_specs=pl.BlockSpec((_G, _D), lambda i: (0, 0)),
        out_shape=jax.ShapeDtypeStruct((_G, _D), jnp.float32),
        scratch_shapes=[
            pltpu.VMEM((_G, 4 * _D), jnp.float32),
            pltpu.VMEM((_G, _D), jnp.float32),
        ],
    )(h1, h2, h3, h4, batch_r, w1, b1, w2, b2, w3, b3, w4, b4)


_CONVS = ("1_l", "2_l", "1_g", "2_g")


def kernel(x, edge_index_1_l, edge_index_2_l, edge_index_1_g, edge_index_2_g,
           batch, params):
    p = params
    eis = (edge_index_1_l, edge_index_2_l, edge_index_1_g, edge_index_2_g)

    npad = _EPAD - _E
    eshape = (_NS, _NSTG, _HLF, _CH)
    srcs, dsts = [], []
    for ei in eis:
        srcs.append(jnp.concatenate([ei[0], jnp.zeros((npad,), jnp.int32)]))
        dsts.append(jnp.concatenate(
            [ei[1], jnp.full((npad,), _DUMP, jnp.int32)]).reshape(eshape))
    dst_all = jnp.stack(dsts)
    src_all = jnp.stack(
        [(s + c * _AROWS).reshape(eshape) for c, s in enumerate(srcs)])
    zeros2d = jnp.zeros((_FCH, _D), jnp.float32)

    l1tab = jnp.concatenate(
        [p["conv1_%s" % t]["nn"]["l1"]["W"] for t in _CONVS], axis=0)
    xpad = jnp.concatenate([x, jnp.zeros((_AROWS - _N,), jnp.int32)])
    xidx = jnp.concatenate([xpad + c * _F for c in range(4)]).reshape(_NW, 10, _FCH)
    Pflat = _sc_gather_p1(l1tab, xidx)
    P = Pflat.reshape(4, _AROWS, _D)

    outs = []
    for L in (1, 2, 3, 4):
        cps = [p["conv%d_%s" % (L, t)] for t in _CONVS]
        scale = jnp.broadcast_to(
            (1.0 + jnp.stack([cp["eps"] for cp in cps]))[:, None], (4, _D))
        b1 = jnp.stack([cp["nn"]["l1"]["b"] for cp in cps])
        W2 = jnp.stack([cp["nn"]["l2"]["W"] for cp in cps])
        b2 = jnp.stack([cp["nn"]["l2"]["b"] for cp in cps])
        m = p["mlp_%d" % L]

        A = _sc_segsum4(Pflat, src_all, dst_all, zeros2d)
        bn = p["bn%d" % L]
        common = (P, A, scale, b1, W2, b2,
                  m["l1"]["W"], m["l1"]["b"].reshape(1, _D),
                  m["l2"]["W"], m["l2"]["b"].reshape(1, _D),
                  bn["gamma"].reshape(1, _D), bn["beta"].reshape(1, _D))
        if L < 4:
            Wn = jnp.stack(
                [p["conv%d_%s" % (L + 1, t)]["nn"]["l1"]["W"] for t in _CONVS])
            h, P = _tc_post_bn_next(*common, Wn)
            Pflat = P.reshape(4 * _AROWS, _D)
        else:
            (h,) = _tc_post_bn_last(*common)
        outs.append(h)

    batch_r = batch.reshape(_PNB, 1, _PB)
    w4pad = jnp.zeros((_D, _D), jnp.float32).at[:, :1].set(p["fc4"]["W"])
    b4pad = jnp.zeros((1, _D), jnp.float32).at[0, 0].set(p["fc4"]["b"][0])
    z = _tc_pool_fc(outs[0], outs[1], outs[2], outs[3], batch_r,
                    p["fc1"]["W"], p["fc1"]["b"].reshape(1, _D),
                    p["fc2"]["W"], p["fc2"]["b"].reshape(1, _D),
                    p["fc3"]["W"], p["fc3"]["b"].reshape(1, _D),
                    w4pad, b4pad)
    return z[:, 0]


# speedup vs baseline: 4.6925x; 1.0145x over previous
import functools

import jax
import jax.numpy as jnp
from jax import lax
from jax.experimental import pallas as pl
from jax.experimental.pallas import tpu as pltpu
from jax.experimental.pallas import tpu_sc as plsc

_N = 10000
_E = 160000
_F = 652
_D = 128
_G = 64

_NC = 2
_NS = 16
_NW = _NC * _NS

_CH = 128
_HLF = 40
_NSTG = 2
_CPT = _NSTG * _HLF
_EPAD = _NS * _CPT * _CH
_FCH = 128
_AROWS = 10240
_DUMP = _N
_RPT = _AROWS // _NS
_ZCH = _RPT // _FCH

_BN = 2000
_NB = _N // _BN
_PB = 1000
_PNB = _N // _PB

_mesh = plsc.VectorSubcoreMesh(core_axis_name="c", subcore_axis_name="s")


@functools.partial(
    pl.kernel,
    out_type=jax.ShapeDtypeStruct((4 * _AROWS, _D), jnp.float32),
    mesh=_mesh,
    scratch_types=[
        pltpu.VMEM((10, _FCH), jnp.int32),
        [pltpu.VMEM((_FCH, _D), jnp.float32)] * 2,
        [pltpu.SemaphoreType.DMA] * 2,
        [pltpu.SemaphoreType.DMA] * 2,
    ],
)
def _sc_gather_p1(tab_hbm, idx_hbm, out_hbm, idx_v, rows, gsems, wsems):
    gwid = lax.axis_index("c") * _NS + lax.axis_index("s")
    pltpu.sync_copy(idx_hbm.at[gwid], idx_v)
    for i in range(2):
        pltpu.async_copy(tab_hbm.at[idx_v.at[i]], rows[i], gsems[i])
    for j in range(10):
        i = j % 2
        pltpu.make_async_copy(tab_hbm.at[idx_v.at[0]], rows[i], gsems[i]).wait()
        pltpu.async_copy(rows[i], out_hbm.at[pl.ds(gwid * 1280 + j * _FCH, _FCH)],
                         wsems[i])
        if j < 8:
            pltpu.make_async_copy(
                rows[i], out_hbm.at[pl.ds(0, _FCH)], wsems[i]).wait()
            pltpu.async_copy(tab_hbm.at[idx_v.at[j + 2]], rows[i], gsems[i])
    for i in range(2):
        pltpu.make_async_copy(rows[i], out_hbm.at[pl.ds(0, _FCH)], wsems[i]).wait()


@functools.partial(
    pl.kernel,
    out_type=jax.ShapeDtypeStruct((4, _AROWS, _D), jnp.float32),
    mesh=_mesh,
    scratch_types=[
        pltpu.VMEM((_HLF, _CH), jnp.int32),
        pltpu.VMEM((_HLF, _CH), jnp.int32),
        [pltpu.VMEM((_CH, _D), jnp.float32)] * 2,
        pltpu.VMEM_SHARED((_AROWS, _D), jnp.float32),
        [pltpu.SemaphoreType.DMA] * 2,
        [pltpu.SemaphoreType.DMA] * 2,
    ],
)
def _sc_segsum4(p_hbm, src_hbm, dst_hbm, zero_hbm, out_hbm,
                src_v, dst_v, bufs, accum, gsems, ssems):
    cid = lax.axis_index("c")
    sid = lax.axis_index("s")
    for k in range(2):
        conv = 2 * cid + k
        pltpu.sync_copy(zero_hbm, accum.at[pl.ds(sid * _RPT, _RPT)])
        plsc.subcore_barrier()
        for half in range(_NSTG):
            pltpu.sync_copy(src_hbm.at[conv, sid, half], src_v)
            pltpu.sync_copy(dst_hbm.at[conv, sid, half], dst_v)
            for i in range(2):
                pltpu.async_copy(p_hbm.at[src_v.at[i]], bufs[i], gsems[i])

            def _body(m, carry):
                j = 2 * m
                jn = lax.min(j + 2, _HLF - 2)
                for i in range(2):
                    pltpu.make_async_copy(p_hbm.at[src_v.at[0]], bufs[i], gsems[i]).wait()
                    pltpu.async_copy(bufs[i], accum.at[dst_v.at[j + i]], ssems[i], add=True)
                    pltpu.make_async_copy(bufs[i], accum.at[dst_v.at[0]], ssems[i]).wait()
                    pltpu.async_copy(p_hbm.at[src_v.at[jn + i]], bufs[i], gsems[i])
                return carry

            lax.fori_loop(0, _HLF // 2, _body, 0)
            for i in range(2):
                pltpu.make_async_copy(p_hbm.at[src_v.at[0]], bufs[i], gsems[i]).wait()
        plsc.subcore_barrier()
        pltpu.sync_copy(accum.at[pl.ds(sid * _RPT, _RPT)],
                        out_hbm.at[conv, pl.ds(sid * _RPT, _RPT)])
        plsc.subcore_barrier()


def _make_post_bn(has_next):
    def body(p_ref, a_ref, sc_ref, b1_ref, w2_ref, b2_ref,
             wm1_ref, bm1_ref, wm2_ref, bm2_ref, g_ref, bt_ref, *rest):
        if has_next:
            wn_ref, h_ref, pn_ref, r_scr, st_scr = rest
        else:
            h_ref, r_scr, st_scr = rest
        ph = pl.program_id(0)
        i = pl.program_id(1)

        @pl.when(ph == 0)
        def _():
            xs = []
            for c in range(4):
                u = jnp.maximum(
                    sc_ref[c][None, :] * p_ref[c] + a_ref[c] + b1_ref[c][None, :], 0.0)
                xc = jnp.dot(u, w2_ref[c], preferred_element_type=jnp.float32)
                xs.append(jnp.maximum(xc + b2_ref[c][None, :], 0.0))
            cat = jnp.concatenate([xs[0], xs[2], xs[1], xs[3]], axis=1)
            y = jnp.maximum(
                jnp.dot(cat, wm1_ref[...], preferred_element_type=jnp.float32)
                + bm1_ref[...], 0.0)
            r = jnp.dot(y, wm2_ref[...], preferred_element_type=jnp.float32) + bm2_ref[...]
            r_scr[pl.ds(i * _BN, _BN), :] = r
            st = jnp.concatenate(
                [jnp.sum(r, axis=0)[None], jnp.sum(r * r, axis=0)[None],
                 jnp.zeros((6, _D), jnp.float32)], axis=0)

            @pl.when(i == 0)
            def _():
                st_scr[...] = st

            @pl.when(i > 0)
            def _():
                st_scr[...] = st_scr[...] + st

        @pl.when(ph == 1)
        def _():
            mean = st_scr[0] * (1.0 / _N)
            var = st_scr[1] * (1.0 / _N) - mean * mean
            inv = lax.rsqrt(var + 1e-5) * g_ref[0]
            r = r_scr[pl.ds(i * _BN, _BN), :]
            h = (r - mean[None, :]) * inv[None, :] + bt_ref[...]
            h_ref[...] = h
            if has_next:
                for c in range(4):
                    pn_ref[c] = jnp.dot(h, wn_ref[c], preferred_element_type=jnp.float32)

    in_specs = [
        pl.BlockSpec((4, _BN, _D), lambda p, i: (0, jnp.where(p == 0, i, _NB - 1), 0)),
        pl.BlockSpec((4, _BN, _D), lambda p, i: (0, jnp.where(p == 0, i, _NB - 1), 0)),
        pl.BlockSpec((4, _D), lambda p, i: (0, 0)),
        pl.BlockSpec((4, _D), lambda p, i: (0, 0)),
        pl.BlockSpec((4, _D, _D), lambda p, i: (0, 0, 0)),
        pl.BlockSpec((4, _D), lambda p, i: (0, 0)),
        pl.BlockSpec((4 * _D, _D), lambda p, i: (0, 0)),
        pl.BlockSpec((1, _D), lambda p, i: (0, 0)),
        pl.BlockSpec((_D, _D), lambda p, i: (0, 0)),
        pl.BlockSpec((1, _D), lambda p, i: (0, 0)),
        pl.BlockSpec((1, _D), lambda p, i: (0, 0)),
        pl.BlockSpec((1, _D), lambda p, i: (0, 0)),
    ]
    out_specs = [pl.BlockSpec((_BN, _D), lambda p, i: (jnp.where(p == 1, i, 0), 0))]
    out_shape = [jax.ShapeDtypeStruct((_N, _D), jnp.float32)]
    if has_next:
        in_specs.append(pl.BlockSpec((4, _D, _D), lambda p, i: (0, 0, 0)))
        out_specs.append(
            pl.BlockSpec((4, _BN, _D), lambda p, i: (0, jnp.where(p == 1, i, 0), 0)))
        out_shape.append(jax.ShapeDtypeStruct((4, _AROWS, _D), jnp.float32))

    def call(*args):
        return pl.pallas_call(
            body,
            grid=(2, _NB),
            in_specs=in_specs,
            out_specs=out_specs,
            out_shape=out_shape,
            scratch_shapes=[
                pltpu.VMEM((_N, _D), jnp.float32),
                pltpu.VMEM((8, _D), jnp.float32),
            ],
        )(*args)

    return call


_tc_post_bn_next = _make_post_bn(True)
_tc_post_bn_last = _make_post_bn(False)


def _pool_fc_kernel(h1_ref, h2_ref, h3_ref, h4_ref, b_ref,
                    w1_ref, b1_ref, w2_ref, b2_ref, w3_ref, b3_ref,
                    w4_ref, b4_ref, out_ref, acc_ref, cnt_ref):
    i = pl.program_id(0)

    @pl.when(i == 0)
    def _():
        acc_ref[...] = jnp.zeros_like(acc_ref)
        cnt_ref[...] = jnp.zeros_like(cnt_ref)

    bb = b_ref[...].reshape(_PB)
    gid = lax.broadcasted_iota(jnp.int32, (_G, _PB), 0)
    oh = (bb[None, :] == gid).astype(jnp.float32)
    cat = jnp.concatenate(
        [h1_ref[...], h2_ref[...], h3_ref[...], h4_ref[...]], axis=1)
    acc_ref[...] = acc_ref[...] + jnp.dot(oh, cat, preferred_element_type=jnp.float32)
    cnt_ref[...] = cnt_ref[...] + jnp.broadcast_to(
        jnp.sum(oh, axis=1, keepdims=True), (_G, _D))

    @pl.when(i == _PNB - 1)
    def _():
        cnt = cnt_ref[...][:, :1]
        pooled = acc_ref[...] / jnp.maximum(cnt, 1.0)
        z = jnp.maximum(
            jnp.dot(pooled, w1_ref[...], preferred_element_type=jnp.float32) + b1_ref[...], 0.0)
        z = jnp.maximum(
            jnp.dot(z, w2_ref[...], preferred_element_type=jnp.float32) + b2_ref[...], 0.0)
        z = jnp.maximum(
            jnp.dot(z, w3_ref[...], preferred_element_type=jnp.float32) + b3_ref[...], 0.0)
        out_ref[...] = jnp.dot(z, w4_ref[...], preferred_element_type=jnp.float32) + b4_ref[...]


def _tc_pool_fc(h1, h2, h3, h4, batch_r, w1, b1, w2, b2, w3, b3, w4, b4):
    hspec = pl.BlockSpec((_PB, _D), lambda i: (i, 0))
    return pl.pallas_call(
        _pool_fc_kernel,
        grid=(_PNB,),
        in_specs=[
            hspec, hspec, hspec, hspec,
            pl.BlockSpec((1, 1, _PB), lambda i: (i, 0, 0)),
            pl.BlockSpec((4 * _D, _D), lambda i: (0, 0)),
            pl.BlockSpec((1, _D), lambda i: (0, 0)),
            pl.BlockSpec((_D, _D), lambda i: (0, 0)),
            pl.BlockSpec((1, _D), lambda i: (0, 0)),
            pl.BlockSpec((_D, _D), lambda i: (0, 0)),
            pl.BlockSpec((1, _D), lambda i: (0, 0)),
            pl.BlockSpec((_D, _D), lambda i: (0, 0)),
            pl.BlockSpec((1, _D), lambda i: (0, 0)),
        ],
        out_specs=pl.BlockSpec((_G, _D), lambda i: (0, 0)),
        out_shape=jax.ShapeDtypeStruct((_G, _D), jnp.float32),
        scratch_shapes=[
            pltpu.VMEM((_G, 4 * _D), jnp.float32),
            pltpu.VMEM((_G, _D), jnp.float32),
        ],
    )(h1, h2, h3, h4, batch_r, w1, b1, w2, b2, w3, b3, w4, b4)


_CONVS = ("1_l", "2_l", "1_g", "2_g")


def kernel(x, edge_index_1_l, edge_index_2_l, edge_index_1_g, edge_index_2_g,
           batch, params):
    p = params
    eis = (edge_index_1_l, edge_index_2_l, edge_index_1_g, edge_index_2_g)

    npad = _EPAD - _E
    eshape = (_NS, _NSTG, _HLF, _CH)
    srcs, dsts = [], []
    for ei in eis:
        srcs.append(jnp.concatenate([ei[0], jnp.zeros((npad,), jnp.int32)]))
        dsts.append(jnp.concatenate(
            [ei[1], jnp.full((npad,), _DUMP, jnp.int32)]).reshape(eshape))
    dst_all = jnp.stack(dsts)
    src_all = jnp.stack(
        [(s + c * _AROWS).reshape(eshape) for c, s in enumerate(srcs)])
    zeros2d = jnp.zeros((_RPT, _D), jnp.float32)

    l1tab = jnp.concatenate(
        [p["conv1_%s" % t]["nn"]["l1"]["W"] for t in _CONVS], axis=0)
    xpad = jnp.concatenate([x, jnp.zeros((_AROWS - _N,), jnp.int32)])
    xidx = jnp.concatenate([xpad + c * _F for c in range(4)]).reshape(_NW, 10, _FCH)
    Pflat = _sc_gather_p1(l1tab, xidx)
    P = Pflat.reshape(4, _AROWS, _D)

    outs = []
    for L in (1, 2, 3, 4):
        cps = [p["conv%d_%s" % (L, t)] for t in _CONVS]
        scale = jnp.broadcast_to(
            (1.0 + jnp.stack([cp["eps"] for cp in cps]))[:, None], (4, _D))
        b1 = jnp.stack([cp["nn"]["l1"]["b"] for cp in cps])
        W2 = jnp.stack([cp["nn"]["l2"]["W"] for cp in cps])
        b2 = jnp.stack([cp["nn"]["l2"]["b"] for cp in cps])
        m = p["mlp_%d" % L]

        A = _sc_segsum4(Pflat, src_all, dst_all, zeros2d)
        bn = p["bn%d" % L]
        common = (P, A, scale, b1, W2, b2,
                  m["l1"]["W"], m["l1"]["b"].reshape(1, _D),
                  m["l2"]["W"], m["l2"]["b"].reshape(1, _D),
                  bn["gamma"].reshape(1, _D), bn["beta"].reshape(1, _D))
        if L < 4:
            Wn = jnp.stack(
                [p["conv%d_%s" % (L + 1, t)]["nn"]["l1"]["W"] for t in _CONVS])
            h, P = _tc_post_bn_next(*common, Wn)
            Pflat = P.reshape(4 * _AROWS, _D)
        else:
            (h,) = _tc_post_bn_last(*common)
        outs.append(h)

    batch_r = batch.reshape(_PNB, 1, _PB)
    w4pad = jnp.zeros((_D, _D), jnp.float32).at[:, :1].set(p["fc4"]["W"])
    b4pad = jnp.zeros((1, _D), jnp.float32).at[0, 0].set(p["fc4"]["b"][0])
    z = _tc_pool_fc(outs[0], outs[1], outs[2], outs[3], batch_r,
                    p["fc1"]["W"], p["fc1"]["b"].reshape(1, _D),
                    p["fc2"]["W"], p["fc2"]["b"].reshape(1, _D),
                    p["fc3"]["W"], p["fc3"]["b"].reshape(1, _D),
                    w4pad, b4pad)
    return z[:, 0]
